# baseline (speedup 1.0000x reference)
import jax
import jax.numpy as jnp
from jax import lax
from jax.experimental import pallas as pl
from jax.experimental.pallas import tpu as pltpu

D_MODEL = 768
E = 64
T = 4096


def _step_math(i, xref, mref, p_ref, i_ref, w_ref, pacc, macc, eacc):
    logits = lax.dot_general(
        w_ref[:], xref[:], (((1,), (1,)), ((), ())),
        preferred_element_type=jnp.float32)
    row = lax.broadcasted_iota(jnp.int32, (E, T), 0)

    m = jnp.max(logits, axis=0, keepdims=True)
    e = jnp.exp(logits - m)
    s = jnp.sum(e, axis=0, keepdims=True)
    r = 1.0 / s
    q = jnp.sum(e * (logits - m), axis=0, keepdims=True)

    i1 = jnp.min(jnp.where(logits == m, row, E), axis=0, keepdims=True)
    lm = jnp.where(row == i1, -jnp.inf, logits)
    m2 = jnp.max(lm, axis=0, keepdims=True)
    i2 = jnp.min(jnp.where(lm == m2, row, E), axis=0, keepdims=True)

    hits = ((row == i1) | (row == i2)).astype(jnp.bfloat16)
    eye_e = jnp.eye(E, dtype=jnp.bfloat16)
    mref[:] = lax.dot_general(
        hits, eye_e, (((0,), (0,)), ((), ())),
        preferred_element_type=jnp.float32)

    p_ref[:] = jnp.transpose(jnp.concatenate([r, jnp.exp(m2 - m) / s], axis=0))
    i_ref[:] = jnp.transpose(jnp.concatenate([i1, i2], axis=0))

    @pl.when(i == 0)
    def _init():
        pacc[:] = jnp.zeros_like(pacc)
        macc[:] = jnp.zeros_like(macc)
        eacc[:] = jnp.zeros_like(eacc)

    pacc[:] += jnp.sum(e * r, axis=1, keepdims=True)
    macc[:] += jnp.sum(hits.astype(jnp.float32), axis=1, keepdims=True)
    eacc[:] += jnp.sum(jnp.log(s) - q * r).reshape(1, 1)


def _router_body(x_hbm, w_ref, p_ref, i_ref, mask_hbm, psum_ref, msum_ref,
                 ent_ref, xbuf0, xbuf1, mbuf0, mbuf1, pacc, macc, eacc,
                 xsem, msem):
    i = pl.program_id(0)
    G = pl.num_programs(0)
    slot = lax.rem(i, 2)
    nslot = lax.rem(i + 1, 2)
    xbufs = (xbuf0, xbuf1)
    mbufs = (mbuf0, mbuf1)

    @pl.when(i == 0)
    def _prologue():
        pltpu.make_async_copy(x_hbm.at[pl.ds(0, T)], xbuf0, xsem.at[0]).start()

    for b in (0, 1):
        @pl.when((i + 1 < G) & (nslot == b))
        def _prefetch(b=b):
            pltpu.make_async_copy(
                x_hbm.at[pl.ds((i + 1) * T, T)], xbufs[b], xsem.at[b]).start()

        @pl.when((i >= 2) & (slot == b))
        def _drain_prev(b=b):
            pltpu.make_async_copy(
                mbufs[b], mask_hbm.at[pl.ds((i - 2) * T, T)], msem.at[b]).wait()

        @pl.when(slot == b)
        def _work(b=b):
            pltpu.make_async_copy(
                x_hbm.at[pl.ds(i * T, T)], xbufs[b], xsem.at[b]).wait()
            _step_math(i, xbufs[b], mbufs[b], p_ref, i_ref, w_ref,
                       pacc, macc, eacc)
            pltpu.make_async_copy(
                mbufs[b], mask_hbm.at[pl.ds(i * T, T)], msem.at[b]).start()

    @pl.when(i == G - 1)
    def _epilogue():
        psum_ref[:] = pacc[:]
        msum_ref[:] = macc[:]
        ent_ref[:] = eacc[:]
        for b in (0, 1):
            @pl.when(nslot == b)
            def _w1(b=b):
                pltpu.make_async_copy(
                    mbufs[b], mask_hbm.at[pl.ds((i - 1) * T, T)], msem.at[b]).wait()

            @pl.when(slot == b)
            def _w2(b=b):
                pltpu.make_async_copy(
                    mbufs[b], mask_hbm.at[pl.ds(i * T, T)], msem.at[b]).wait()


def kernel(x, W, temp):
    B, S, D = x.shape
    N = B * S
    t = jnp.clip(temp, 0.1, 5.0)
    w = W / t
    xf = x.reshape(N, D)
    grid = N // T

    p_pair, i_pair, mask, psum, msum, ent = pl.pallas_call(
        _router_body,
        grid=(grid,),
        in_specs=[
            pl.BlockSpec(memory_space=pl.ANY),
            pl.BlockSpec((E, D), lambda i: (0, 0)),
        ],
        out_specs=[
            pl.BlockSpec((T, 2), lambda i: (i, 0)),
            pl.BlockSpec((T, 2), lambda i: (i, 0)),
            pl.BlockSpec(memory_space=pl.ANY),
            pl.BlockSpec((E, 1), lambda i: (0, 0)),
            pl.BlockSpec((E, 1), lambda i: (0, 0)),
            pl.BlockSpec((1, 1), lambda i: (0, 0)),
        ],
        out_shape=[
            jax.ShapeDtypeStruct((N, 2), jnp.float32),
            jax.ShapeDtypeStruct((N, 2), jnp.int32),
            jax.ShapeDtypeStruct((N, E), jnp.float32),
            jax.ShapeDtypeStruct((E, 1), jnp.float32),
            jax.ShapeDtypeStruct((E, 1), jnp.float32),
            jax.ShapeDtypeStruct((1, 1), jnp.float32),
        ],
        scratch_shapes=[
            pltpu.VMEM((T, D_MODEL), jnp.float32),
            pltpu.VMEM((T, D_MODEL), jnp.float32),
            pltpu.VMEM((T, E), jnp.float32),
            pltpu.VMEM((T, E), jnp.float32),
            pltpu.VMEM((E, 1), jnp.float32),
            pltpu.VMEM((E, 1), jnp.float32),
            pltpu.VMEM((1, 1), jnp.float32),
            pltpu.SemaphoreType.DMA((2,)),
            pltpu.SemaphoreType.DMA((2,)),
        ],
    )(xf, w)

    expert_probs = p_pair.reshape(B, S, 2)
    expert_indices = i_pair.reshape(B, S, 2)
    expert_mask = mask.reshape(B, S, E)

    denom = jnp.float32(N)
    importance = psum[:, 0] / denom
    load = msum[:, 0] / (denom + 1e-6)
    aux_load_loss = jnp.sum(importance * load) * E * 0.01
    router_entropy = (ent[0, 0] / denom) * 0.01
    return expert_probs, expert_indices, expert_mask, aux_load_loss, router_entropy


# speedup vs baseline: 1.1383x; 1.1383x over previous
import jax
import jax.numpy as jnp
from jax import lax
from jax.experimental import pallas as pl

D_MODEL = 768
E = 64


def _router_body(x_ref, w_ref, p_ref, i_ref, mask_ref, psum_ref, msum_ref, ent_ref):
    T = x_ref.shape[0]
    logits = lax.dot_general(
        w_ref[:], x_ref[:], (((1,), (1,)), ((), ())),
        preferred_element_type=jnp.float32)
    row = lax.broadcasted_iota(jnp.int32, (E, T), 0)

    m = jnp.max(logits, axis=0, keepdims=True)
    e = jnp.exp(logits - m)
    s = jnp.sum(e, axis=0, keepdims=True)
    r = 1.0 / s
    q = jnp.sum(e * (logits - m), axis=0, keepdims=True)

    i1 = jnp.min(jnp.where(logits == m, row, E), axis=0, keepdims=True)
    lm = jnp.where(row == i1, -jnp.inf, logits)
    m2 = jnp.max(lm, axis=0, keepdims=True)
    i2 = jnp.min(jnp.where(lm == m2, row, E), axis=0, keepdims=True)

    hits = ((row == i1) | (row == i2)).astype(jnp.float32)
    mask_ref[:] = jnp.transpose(hits)

    p_ref[:] = jnp.transpose(jnp.concatenate([r, jnp.exp(m2 - m) / s], axis=0))
    i_ref[:] = jnp.transpose(jnp.concatenate([i1, i2], axis=0))

    @pl.when(pl.program_id(0) == 0)
    def _init():
        psum_ref[:] = jnp.zeros_like(psum_ref)
        msum_ref[:] = jnp.zeros_like(msum_ref)
        ent_ref[:] = jnp.zeros_like(ent_ref)

    psum_ref[:] += jnp.sum(e * r, axis=1, keepdims=True)
    msum_ref[:] += jnp.sum(hits, axis=1, keepdims=True)
    ent_ref[:] += jnp.sum(jnp.log(s) - q * r).reshape(1, 1)


def kernel(x, W, temp):
    B, S, D = x.shape
    N = B * S
    t = jnp.clip(temp, 0.1, 5.0)
    w = W / t
    xf = x.reshape(N, D)
    T = 4096
    grid = N // T

    p_pair, i_pair, mask, psum, msum, ent = pl.pallas_call(
        _router_body,
        grid=(grid,),
        in_specs=[
            pl.BlockSpec((T, D), lambda i: (i, 0)),
            pl.BlockSpec((E, D), lambda i: (0, 0)),
        ],
        out_specs=[
            pl.BlockSpec((T, 2), lambda i: (i, 0)),
            pl.BlockSpec((T, 2), lambda i: (i, 0)),
            pl.BlockSpec((T, E), lambda i: (i, 0)),
            pl.BlockSpec((E, 1), lambda i: (0, 0)),
            pl.BlockSpec((E, 1), lambda i: (0, 0)),
            pl.BlockSpec((1, 1), lambda i: (0, 0)),
        ],
        out_shape=[
            jax.ShapeDtypeStruct((N, 2), jnp.float32),
            jax.ShapeDtypeStruct((N, 2), jnp.int32),
            jax.ShapeDtypeStruct((N, E), jnp.float32),
            jax.ShapeDtypeStruct((E, 1), jnp.float32),
            jax.ShapeDtypeStruct((E, 1), jnp.float32),
            jax.ShapeDtypeStruct((1, 1), jnp.float32),
        ],
    )(xf, w)

    expert_probs = p_pair.reshape(B, S, 2)
    expert_indices = i_pair.reshape(B, S, 2)
    expert_mask = mask.reshape(B, S, E)

    denom = jnp.float32(N)
    importance = psum[:, 0] / denom
    load = msum[:, 0] / (denom + 1e-6)
    aux_load_loss = jnp.sum(importance * load) * E * 0.01
    router_entropy = (ent[0, 0] / denom) * 0.01
    return expert_probs, expert_indices, expert_mask, aux_load_loss, router_entropy
